# Initial kernel scaffold; baseline (speedup 1.0000x reference)
import jax
import jax.numpy as jnp
from jax.experimental import pallas as pl


def kernel(edge_index, Gu, Gi, Gut, Git):
    raise NotImplementedError("write your pallas kernel here")


# speedup vs baseline: 6.5016x; 6.5016x over previous
import jax
import jax.numpy as jnp
from jax import lax
from jax.experimental import pallas as pl
from jax.experimental.pallas import tpu as pltpu
from jax.experimental.pallas import tpu_sc as plsc

N_NODES = 50000
H = 32
NC, NS = 2, 16
ROWS_PER_TILE = 3136
N_PAD = NS * ROWS_PER_TILE
E = 800000
E_PER_TILE = E // NS
CHUNK = 128
N_FULL = E_PER_TILE // CHUNK
TAIL = E_PER_TILE - N_FULL * CHUNK
ZROWS = ROWS_PER_TILE // 4


def _layer_body(y_hbm, src_hbm, dst_hbm, out_hbm,
                sidx, didx, sidx_t, didx_t, rows, zbuf, acc, sem):
    c = lax.axis_index("c")
    s = lax.axis_index("s")

    def zrow(r, _):
        zbuf[r, pl.ds(0, 16)] = jnp.zeros((16,), jnp.float32)
        zbuf[r, pl.ds(16, 16)] = jnp.zeros((16,), jnp.float32)
        return 0
    lax.fori_loop(0, ZROWS, zrow, 0)
    base_row = s * ROWS_PER_TILE
    for i in range(ROWS_PER_TILE // ZROWS):
        pltpu.sync_copy(zbuf, acc.at[pl.ds(base_row + i * ZROWS, ZROWS)])
    plsc.subcore_barrier()

    off = c * N_PAD
    ebase = s * E_PER_TILE

    def process(base, si, di, k):
        pltpu.sync_copy(src_hbm.at[pl.ds(base, k)], si)
        pltpu.sync_copy(dst_hbm.at[pl.ds(base, k)], di)
        for j in range(k // 16):
            si[pl.ds(j * 16, 16)] = si[pl.ds(j * 16, 16)] + off
        pltpu.async_copy(y_hbm.at[si], rows.at[pl.ds(0, k)], sem).wait()
        pltpu.sync_copy(rows.at[pl.ds(0, k)], acc.at[di], add=True)

    def chunk_body(i, _):
        process(ebase + i * CHUNK, sidx, didx, CHUNK)
        return 0
    lax.fori_loop(0, N_FULL, chunk_body, 0)
    process(ebase + N_FULL * CHUNK, sidx_t, didx_t, TAIL)

    plsc.subcore_barrier()
    pltpu.sync_copy(acc.at[pl.ds(base_row, ROWS_PER_TILE)],
                    out_hbm.at[pl.ds(c * N_PAD + base_row, ROWS_PER_TILE)])


_layer = pl.kernel(
    _layer_body,
    out_type=jax.ShapeDtypeStruct((NC * N_PAD, H), jnp.float32),
    mesh=plsc.VectorSubcoreMesh(core_axis_name="c", subcore_axis_name="s"),
    scratch_types=[
        pltpu.VMEM((CHUNK,), jnp.int32),
        pltpu.VMEM((CHUNK,), jnp.int32),
        pltpu.VMEM((TAIL,), jnp.int32),
        pltpu.VMEM((TAIL,), jnp.int32),
        pltpu.VMEM((CHUNK, H), jnp.float32),
        pltpu.VMEM((ZROWS, H), jnp.float32),
        pltpu.VMEM_SHARED((N_PAD, H), jnp.float32),
        pltpu.SemaphoreType.DMA,
    ],
    compiler_params=pltpu.CompilerParams(use_tc_tiling_on_sc=False),
)


def _to_split(x):
    xs = jnp.stack([x[:, :H], x[:, H:]], axis=0)
    return jnp.pad(xs, ((0, 0), (0, N_PAD - N_NODES), (0, 0))).reshape(
        NC * N_PAD, H)


def _from_split(a):
    a = a.reshape(NC, N_PAD, H)[:, :N_NODES, :]
    return jnp.concatenate([a[0], a[1]], axis=1)


def kernel(edge_index, Gu, Gi, Gut, Git):
    src = edge_index[0]
    dst = edge_index[1]
    x0 = jnp.concatenate([Gu + Gut, Gi + Git], axis=0)

    ones2 = jnp.ones((NC * N_PAD, H), jnp.float32)
    deg_dst = _layer(ones2, src, dst).reshape(NC, N_PAD, H)[0, :N_NODES, 0]
    deg_src = _layer(ones2, dst, src).reshape(NC, N_PAD, H)[0, :N_NODES, 0]
    inv_s = lax.rsqrt(jnp.maximum(deg_src, 1.0))
    inv_d = lax.rsqrt(jnp.maximum(deg_dst, 1.0))
    inv_s2 = jnp.tile(jnp.pad(inv_s, (0, N_PAD - N_NODES)), NC)[:, None]
    inv_d2 = jnp.tile(jnp.pad(inv_d, (0, N_PAD - N_NODES)), NC)[:, None]

    x2 = _to_split(x0)
    out2 = x2
    for l in range(3):
        a2 = _layer(x2 * inv_s2, src, dst)
        x2 = a2 * inv_d2
        out2 = out2 + x2 * (1.0 / (l + 2))
    return _from_split(out2)
